# baseline (speedup 1.0000x reference)
import functools

import jax
import jax.numpy as jnp
from jax import lax
from jax.experimental import pallas as pl
from jax.experimental.pallas import tpu as pltpu
from jax.experimental.pallas import tpu_sc as plsc

D = 128
NC, NS = 2, 16
NW = NC * NS
BPG = 8


def _make_gather(batch: int, hist: int):
  b_per_w = batch // NW
  ngroup = b_per_w // BPG
  mesh = plsc.VectorSubcoreMesh(core_axis_name="c", subcore_axis_name="s")

  @functools.partial(
      pl.kernel,
      mesh=mesh,
      compiler_params=pltpu.CompilerParams(use_tc_tiling_on_sc=True),
      out_type=jax.ShapeDtypeStruct((batch, hist, D), jnp.float32),
      scratch_types=[
          pltpu.VMEM((b_per_w, hist), jnp.int32),
          pltpu.VMEM((BPG, hist, D), jnp.float32),
          pltpu.VMEM((BPG, hist, D), jnp.float32),
          pltpu.SemaphoreType.DMA,
          pltpu.SemaphoreType.DMA,
          pltpu.SemaphoreType.DMA,
          pltpu.SemaphoreType.DMA,
      ],
  )
  def gather(x_hbm, table_hbm, out_hbm, idx_v, rows0, rows1, g0, g1, w0, w1):
    wid = lax.axis_index("s") * NC + lax.axis_index("c")
    base = wid * b_per_w
    rows = (rows0, rows1)
    gsem = (g0, g1)
    wsem = (w0, w1)
    pltpu.sync_copy(x_hbm.at[pl.ds(base, b_per_w)], idx_v)

    def start_group(g, buf):
      for j in range(BPG):
        pltpu.async_copy(
            table_hbm.at[idx_v.at[g * BPG + j]],
            rows[buf].at[j],
            gsem[buf],
        )

    def wait_group(g, buf):
      for j in range(BPG):
        pltpu.make_async_copy(
            table_hbm.at[idx_v.at[g * BPG + j]],
            rows[buf].at[j],
            gsem[buf],
        ).wait()

    def start_write(g, buf):
      pltpu.async_copy(rows[buf], out_hbm.at[pl.ds(base + g * BPG, BPG)],
                       wsem[buf])

    def wait_write(g, buf):
      pltpu.make_async_copy(rows[buf], out_hbm.at[pl.ds(base + g * BPG, BPG)],
                            wsem[buf]).wait()

    start_group(0, 0)
    start_group(1, 1)

    def outer(o, carry):
      for buf in range(2):
        g = o * 2 + buf
        wait_group(g, buf)
        start_write(g, buf)
      for buf in range(2):
        g = o * 2 + buf
        wait_write(g, buf)
        start_group(g + 2, buf)
      return carry

    lax.fori_loop(0, ngroup // 2 - 1, outer, 0)

    for buf in range(2):
      g = ngroup - 2 + buf
      wait_group(g, buf)
      start_write(g, buf)
    for buf in range(2):
      g = ngroup - 2 + buf
      wait_write(g, buf)

  return gather


_gather = _make_gather(4096, 50)


def kernel(x, table):
  return _gather(x, table)


# speedup vs baseline: 1.6894x; 1.6894x over previous
import functools

import jax
import jax.numpy as jnp
from jax import lax
from jax.experimental import pallas as pl
from jax.experimental.pallas import tpu as pltpu
from jax.experimental.pallas import tpu_sc as plsc

D = 128
NC, NS = 2, 16
NW = NC * NS


def _make_gather(batch: int, hist: int):
  bw = batch // NW
  mesh = plsc.VectorSubcoreMesh(core_axis_name="c", subcore_axis_name="s")

  @functools.partial(
      pl.kernel,
      mesh=mesh,
      out_type=jax.ShapeDtypeStruct((hist, batch, D), jnp.float32),
      scratch_types=[
          pltpu.VMEM((hist, bw), jnp.int32),
          pltpu.VMEM((bw, D), jnp.float32),
          pltpu.VMEM((bw, D), jnp.float32),
          pltpu.SemaphoreType.DMA,
          pltpu.SemaphoreType.DMA,
          pltpu.SemaphoreType.DMA,
          pltpu.SemaphoreType.DMA,
      ],
  )
  def gather(xt_hbm, table_hbm, out_hbm, idx_v, rows0, rows1, g0, g1, w0, w1):
    wid = lax.axis_index("s") * NC + lax.axis_index("c")
    base = wid * bw
    rows = (rows0, rows1)
    gsem = (g0, g1)
    wsem = (w0, w1)
    pltpu.sync_copy(xt_hbm.at[:, pl.ds(base, bw)], idx_v)

    def start_g(h, buf):
      pltpu.async_copy(table_hbm.at[idx_v.at[h]], rows[buf], gsem[buf])

    def wait_g(h, buf):
      pltpu.make_async_copy(table_hbm.at[idx_v.at[h]], rows[buf],
                            gsem[buf]).wait()

    def start_w(h, buf):
      pltpu.async_copy(rows[buf], out_hbm.at[h, pl.ds(base, bw)], wsem[buf])

    def wait_w(h, buf):
      pltpu.make_async_copy(rows[buf], out_hbm.at[h, pl.ds(base, bw)],
                            wsem[buf]).wait()

    start_g(0, 0)
    start_g(1, 1)

    def outer(o, carry):
      for buf in range(2):
        h = o * 2 + buf
        wait_g(h, buf)
        start_w(h, buf)
      for buf in range(2):
        h = o * 2 + buf
        wait_w(h, buf)
        start_g(h + 2, buf)
      return carry

    lax.fori_loop(0, hist // 2 - 1, outer, 0)

    for buf in range(2):
      h = hist - 2 + buf
      wait_g(h, buf)
      start_w(h, buf)
    for buf in range(2):
      wait_w(hist - 2 + buf, buf)

  return gather


_gather = _make_gather(4096, 50)


def kernel(x, table):
  xt = jnp.swapaxes(x, 0, 1)
  out_t = _gather(xt, table)
  return jnp.swapaxes(out_t, 0, 1)


# speedup vs baseline: 1.8147x; 1.0742x over previous
import functools

import jax
import jax.numpy as jnp
from jax import lax
from jax.experimental import pallas as pl
from jax.experimental.pallas import tpu as pltpu
from jax.experimental.pallas import tpu_sc as plsc

D = 128
NC, NS = 2, 16
NW = NC * NS


def _make_gather(batch: int, hist: int):
  bw = batch // NW
  mesh = plsc.VectorSubcoreMesh(core_axis_name="c", subcore_axis_name="s")

  @functools.partial(
      pl.kernel,
      mesh=mesh,
      out_type=jax.ShapeDtypeStruct((hist, batch, D), jnp.float32),
      scratch_types=[
          pltpu.VMEM((hist, bw), jnp.int32),
          pltpu.VMEM((bw, D), jnp.float32),
          pltpu.VMEM((bw, D), jnp.float32),
          pltpu.VMEM((bw, D), jnp.float32),
          pltpu.VMEM((bw, D), jnp.float32),
          pltpu.SemaphoreType.DMA,
          pltpu.SemaphoreType.DMA,
          pltpu.SemaphoreType.DMA,
          pltpu.SemaphoreType.DMA,
          pltpu.SemaphoreType.DMA,
          pltpu.SemaphoreType.DMA,
          pltpu.SemaphoreType.DMA,
          pltpu.SemaphoreType.DMA,
      ],
  )
  def gather(xt_hbm, table_hbm, out_hbm, idx_v,
             r0, r1, r2, r3, g0, g1, g2, g3, w0, w1, w2, w3):
    wid = lax.axis_index("s") * NC + lax.axis_index("c")
    base = wid * bw
    rows = (r0, r1, r2, r3)
    gsem = (g0, g1, g2, g3)
    wsem = (w0, w1, w2, w3)
    nbuf = 4
    pltpu.sync_copy(xt_hbm.at[:, pl.ds(base, bw)], idx_v)

    def start_g(h, buf):
      pltpu.async_copy(table_hbm.at[idx_v.at[h]], rows[buf], gsem[buf])

    def wait_g(h, buf):
      pltpu.make_async_copy(table_hbm.at[idx_v.at[h]], rows[buf],
                            gsem[buf]).wait()

    def start_w(h, buf):
      pltpu.async_copy(rows[buf], out_hbm.at[h, pl.ds(base, bw)], wsem[buf])

    def wait_w(h, buf):
      pltpu.make_async_copy(rows[buf], out_hbm.at[h, pl.ds(base, bw)],
                            wsem[buf]).wait()

    for h in range(nbuf):
      start_g(h, h)

    nloop = hist // nbuf - 1
    def outer(o, carry):
      for buf in range(nbuf):
        h = o * nbuf + buf
        wait_g(h, buf)
        start_w(h, buf)
      for buf in range(nbuf):
        h = o * nbuf + buf
        wait_w(h, buf)
        start_g(h + nbuf, buf)
      return carry

    lax.fori_loop(0, nloop, outer, 0)

    tail0 = nloop * nbuf
    for h in range(tail0, tail0 + nbuf):
      wait_g(h, h % nbuf)
      start_w(h, h % nbuf)
    for h in range(tail0, tail0 + nbuf):
      wait_w(h, h % nbuf)
      if h + nbuf < hist:
        start_g(h + nbuf, h % nbuf)
    for h in range(tail0 + nbuf, hist):
      wait_g(h, h % nbuf)
      start_w(h, h % nbuf)
    for h in range(tail0 + nbuf, hist):
      wait_w(h, h % nbuf)

  return gather


_gather = _make_gather(4096, 50)


def kernel(x, table):
  xt = jnp.swapaxes(x, 0, 1)
  out_t = _gather(xt, table)
  return jnp.swapaxes(out_t, 0, 1)


# speedup vs baseline: 1.8375x; 1.0126x over previous
import functools

import jax
import jax.numpy as jnp
from jax import lax
from jax.experimental import pallas as pl
from jax.experimental.pallas import tpu as pltpu
from jax.experimental.pallas import tpu_sc as plsc

D = 128
NC, NS = 2, 16
NW = NC * NS


def _make_gather(batch: int, hist: int):
  bw = batch // NW
  mesh = plsc.VectorSubcoreMesh(core_axis_name="c", subcore_axis_name="s")

  @functools.partial(
      pl.kernel,
      mesh=mesh,
      out_type=jax.ShapeDtypeStruct((hist, batch, D), jnp.float32),
      scratch_types=[
          pltpu.VMEM((hist, bw), jnp.int32),
          pltpu.VMEM((bw, D), jnp.float32),
          pltpu.VMEM((bw, D), jnp.float32),
          pltpu.VMEM((bw, D), jnp.float32),
          pltpu.VMEM((bw, D), jnp.float32),
          pltpu.VMEM((bw, D), jnp.float32),
          pltpu.VMEM((bw, D), jnp.float32),
          pltpu.SemaphoreType.DMA,
          pltpu.SemaphoreType.DMA,
          pltpu.SemaphoreType.DMA,
          pltpu.SemaphoreType.DMA,
          pltpu.SemaphoreType.DMA,
          pltpu.SemaphoreType.DMA,
          pltpu.SemaphoreType.DMA,
          pltpu.SemaphoreType.DMA,
          pltpu.SemaphoreType.DMA,
          pltpu.SemaphoreType.DMA,
          pltpu.SemaphoreType.DMA,
          pltpu.SemaphoreType.DMA,
      ],
  )
  def gather(xt_hbm, table_hbm, out_hbm, idx_v,
             r0, r1, r2, r3, r4, r5, g0, g1, g2, g3, g4, g5,
             w0, w1, w2, w3, w4, w5):
    wid = lax.axis_index("s") * NC + lax.axis_index("c")
    base = wid * bw
    rows = (r0, r1, r2, r3, r4, r5)
    gsem = (g0, g1, g2, g3, g4, g5)
    wsem = (w0, w1, w2, w3, w4, w5)
    nbuf = 6
    pltpu.sync_copy(xt_hbm.at[:, pl.ds(base, bw)], idx_v)

    def start_g(h, buf):
      pltpu.async_copy(table_hbm.at[idx_v.at[h]], rows[buf], gsem[buf])

    def wait_g(h, buf):
      pltpu.make_async_copy(table_hbm.at[idx_v.at[h]], rows[buf],
                            gsem[buf]).wait()

    def start_w(h, buf):
      pltpu.async_copy(rows[buf], out_hbm.at[h, pl.ds(base, bw)], wsem[buf])

    def wait_w(h, buf):
      pltpu.make_async_copy(rows[buf], out_hbm.at[h, pl.ds(base, bw)],
                            wsem[buf]).wait()

    for h in range(nbuf):
      start_g(h, h)

    nloop = hist // nbuf - 1
    def outer(o, carry):
      for buf in range(nbuf):
        h = o * nbuf + buf
        wait_g(h, buf)
        start_w(h, buf)
      for buf in range(nbuf):
        h = o * nbuf + buf
        wait_w(h, buf)
        start_g(h + nbuf, buf)
      return carry

    lax.fori_loop(0, nloop, outer, 0)

    tail0 = nloop * nbuf
    for h in range(tail0, tail0 + nbuf):
      wait_g(h, h % nbuf)
      start_w(h, h % nbuf)
    for h in range(tail0, tail0 + nbuf):
      wait_w(h, h % nbuf)
      if h + nbuf < hist:
        start_g(h + nbuf, h % nbuf)
    for h in range(tail0 + nbuf, hist):
      wait_g(h, h % nbuf)
      start_w(h, h % nbuf)
    for h in range(tail0 + nbuf, hist):
      wait_w(h, h % nbuf)

  return gather


_gather = _make_gather(4096, 50)


def kernel(x, table):
  xt = jnp.swapaxes(x, 0, 1)
  out_t = _gather(xt, table)
  return jnp.swapaxes(out_t, 0, 1)


# speedup vs baseline: 1.8465x; 1.0049x over previous
import functools

import jax
import jax.numpy as jnp
from jax import lax
from jax.experimental import pallas as pl
from jax.experimental.pallas import tpu as pltpu
from jax.experimental.pallas import tpu_sc as plsc

D = 128
NC, NS = 2, 16
NW = NC * NS


def _make_gather(batch: int, hist: int):
  bw = batch // NW
  mesh = plsc.VectorSubcoreMesh(core_axis_name="c", subcore_axis_name="s")

  @functools.partial(
      pl.kernel,
      mesh=mesh,
      out_type=jax.ShapeDtypeStruct((hist, batch, D), jnp.float32),
      scratch_types=[
          pltpu.VMEM((hist, bw), jnp.int32),
          pltpu.VMEM((bw, D), jnp.float32),
          pltpu.VMEM((bw, D), jnp.float32),
          pltpu.VMEM((bw, D), jnp.float32),
          pltpu.VMEM((bw, D), jnp.float32),
          pltpu.VMEM((bw, D), jnp.float32),
          pltpu.SemaphoreType.DMA,
          pltpu.SemaphoreType.DMA,
          pltpu.SemaphoreType.DMA,
          pltpu.SemaphoreType.DMA,
          pltpu.SemaphoreType.DMA,
          pltpu.SemaphoreType.DMA,
          pltpu.SemaphoreType.DMA,
          pltpu.SemaphoreType.DMA,
          pltpu.SemaphoreType.DMA,
          pltpu.SemaphoreType.DMA,
      ],
  )
  def gather(xt_hbm, table_hbm, out_hbm, idx_v,
             r0, r1, r2, r3, r4, g0, g1, g2, g3, g4,
             w0, w1, w2, w3, w4):
    wid = lax.axis_index("s") * NC + lax.axis_index("c")
    base = wid * bw
    rows = (r0, r1, r2, r3, r4)
    gsem = (g0, g1, g2, g3, g4)
    wsem = (w0, w1, w2, w3, w4)
    nbuf = 5
    pltpu.sync_copy(xt_hbm.at[:, pl.ds(base, bw)], idx_v)

    def start_g(h, buf):
      pltpu.async_copy(table_hbm.at[idx_v.at[h]], rows[buf], gsem[buf])

    def wait_g(h, buf):
      pltpu.make_async_copy(table_hbm.at[idx_v.at[h]], rows[buf],
                            gsem[buf]).wait()

    def start_w(h, buf):
      pltpu.async_copy(rows[buf], out_hbm.at[h, pl.ds(base, bw)], wsem[buf])

    def wait_w(h, buf):
      pltpu.make_async_copy(rows[buf], out_hbm.at[h, pl.ds(base, bw)],
                            wsem[buf]).wait()

    for h in range(nbuf):
      start_g(h, h)

    nloop = hist // nbuf - 1
    def outer(o, carry):
      for buf in range(nbuf):
        h = o * nbuf + buf
        wait_g(h, buf)
        start_w(h, buf)
      for buf in range(nbuf):
        h = o * nbuf + buf
        wait_w(h, buf)
        start_g(h + nbuf, buf)
      return carry

    lax.fori_loop(0, nloop, outer, 0)

    tail0 = nloop * nbuf
    for h in range(tail0, tail0 + nbuf):
      wait_g(h, h % nbuf)
      start_w(h, h % nbuf)
    for h in range(tail0, tail0 + nbuf):
      wait_w(h, h % nbuf)
      if h + nbuf < hist:
        start_g(h + nbuf, h % nbuf)
    for h in range(tail0 + nbuf, hist):
      wait_g(h, h % nbuf)
      start_w(h, h % nbuf)
    for h in range(tail0 + nbuf, hist):
      wait_w(h, h % nbuf)

  return gather


_gather = _make_gather(4096, 50)


def kernel(x, table):
  xt = jnp.swapaxes(x, 0, 1)
  out_t = _gather(xt, table)
  return jnp.swapaxes(out_t, 0, 1)
